# baseline (speedup 1.0000x reference)
import jax
import jax.numpy as jnp
from jax.experimental import pallas as pl

QB = 512
LPAD = 1024


def _dist_argmin_kernel(x_ref, cm_ref, out_ref):
    xb = x_ref[...]
    feac = jnp.concatenate(
        [xb, jnp.ones((xb.shape[0], 1), dtype=xb.dtype)], axis=1)
    nrm = jnp.sqrt(jnp.sum(feac * feac, axis=1, keepdims=True))
    fea = feac / nrm
    d1 = feac.shape[1]
    dot = jnp.dot(fea, cm_ref[0:d1, :], preferred_element_type=jnp.float32)
    scores = cm_ref[d1:d1 + 1, :] - 2.0 * dot
    pred = jnp.argmin(scores, axis=1).astype(jnp.int32)
    out_ref[0, 0, :] = pred


def kernel(x, initc, labelset):
    q, d = x.shape
    l = labelset.shape[0]
    centers = jnp.take(initc, labelset, axis=0)
    cn = jnp.sum(centers * centers, axis=1)
    cmat = jnp.full((d + 2, LPAD), jnp.inf, dtype=jnp.float32)
    cmat = cmat.at[:d + 1, :].set(0.0)
    cmat = cmat.at[:d + 1, :l].set(centers.T)
    cmat = cmat.at[d + 1, :l].set(cn)

    nq = q // QB
    pred = pl.pallas_call(
        _dist_argmin_kernel,
        grid=(nq,),
        in_specs=[
            pl.BlockSpec((QB, d), lambda i: (i, 0)),
            pl.BlockSpec((d + 2, LPAD), lambda i: (0, 0)),
        ],
        out_specs=pl.BlockSpec((1, 1, QB), lambda i: (i, 0, 0)),
        out_shape=jax.ShapeDtypeStruct((nq, 1, QB), jnp.int32),
    )(x, cmat)
    pred = pred.reshape(q)
    return jnp.take(labelset, pred, axis=0)


# speedup vs baseline: 1.0888x; 1.0888x over previous
import jax
import jax.numpy as jnp
from jax.experimental import pallas as pl

QB = 512
LPAD = 1024


def _dist_argmin_kernel(x_ref, cm_ref, cn_ref, out_ref):
    xb = x_ref[...]
    feac = jnp.concatenate(
        [xb, jnp.ones((xb.shape[0], 1), dtype=xb.dtype)], axis=1)
    nrm = jnp.sqrt(jnp.sum(feac * feac, axis=1, keepdims=True))
    fea = feac / nrm
    dot = jnp.dot(fea, cm_ref[...], preferred_element_type=jnp.float32)
    scores = cn_ref[0:1, :] - 2.0 * dot
    pred = jnp.argmin(scores, axis=1).astype(jnp.int32)
    out_ref[0, 0, :] = pred


def kernel(x, initc, labelset):
    q, d = x.shape
    l = labelset.shape[0]
    centers = jnp.take(initc, labelset, axis=0)
    cn = jnp.sum(centers * centers, axis=1)
    cmat = jnp.zeros((d + 1, LPAD), dtype=jnp.float32)
    cmat = cmat.at[:, :l].set(centers.T)
    cnrow = jnp.full((8, LPAD), jnp.inf, dtype=jnp.float32)
    cnrow = cnrow.at[0, :l].set(cn)

    nq = q // QB
    pred = pl.pallas_call(
        _dist_argmin_kernel,
        grid=(nq,),
        in_specs=[
            pl.BlockSpec((QB, d), lambda i: (i, 0)),
            pl.BlockSpec((d + 1, LPAD), lambda i: (0, 0)),
            pl.BlockSpec((8, LPAD), lambda i: (0, 0)),
        ],
        out_specs=pl.BlockSpec((1, 1, QB), lambda i: (i, 0, 0)),
        out_shape=jax.ShapeDtypeStruct((nq, 1, QB), jnp.int32),
    )(x, cmat, cnrow)
    pred = pred.reshape(q)
    return jnp.take(labelset, pred, axis=0)


# speedup vs baseline: 2.0136x; 1.8493x over previous
import functools

import jax
import jax.numpy as jnp
from jax.experimental import pallas as pl
from jax.experimental.pallas import tpu as pltpu

QB = 512
LPAD = 1024


def _cdcl_kernel(nvalid, x_ref, c_ref, lab_ref, out_ref, cm_s, cn_s, code_s):
    @pl.when(pl.program_id(0) == 0)
    def _prep():
        lab = lab_ref[...]
        ksub = jax.lax.broadcasted_iota(
            jnp.int32, (c_ref.shape[0], LPAD), 0)
        oht = (ksub == lab).astype(jnp.float32)
        cm = jax.lax.dot_general(
            c_ref[...], oht, (((0,), (0,)), ((), ())),
            precision=jax.lax.Precision.HIGHEST,
            preferred_element_type=jnp.float32)
        cm_s[...] = cm
        cn = jnp.sum(cm * cm, axis=0, keepdims=True)
        lane = jax.lax.broadcasted_iota(jnp.int32, (1, LPAD), 1)
        cn_s[...] = jnp.where(lane < nvalid, cn, jnp.inf)
        code_s[...] = lane * 2048 + lab

    xb = x_ref[...]
    feac = jnp.concatenate(
        [xb, jnp.ones((xb.shape[0], 1), dtype=xb.dtype)], axis=1)
    nrm = jnp.sqrt(jnp.sum(feac * feac, axis=1, keepdims=True))
    fea = feac / nrm
    dot = jnp.dot(fea, cm_s[...], preferred_element_type=jnp.float32)
    scores = cn_s[...] - 2.0 * dot
    m = jnp.min(scores, axis=1, keepdims=True)
    sel = jnp.where(scores == m, code_s[...], jnp.int32(2**30))
    out_ref[0, 0, :] = jnp.min(sel, axis=1) & 2047


def kernel(x, initc, labelset):
    q, d = x.shape
    l = labelset.shape[0]
    lab2d = jnp.zeros((1, LPAD), dtype=jnp.int32).at[0, :l].set(labelset)

    nq = q // QB
    labels = pl.pallas_call(
        functools.partial(_cdcl_kernel, l),
        grid=(nq,),
        in_specs=[
            pl.BlockSpec((QB, d), lambda i: (i, 0)),
            pl.BlockSpec(initc.shape, lambda i: (0, 0)),
            pl.BlockSpec((1, LPAD), lambda i: (0, 0)),
        ],
        out_specs=pl.BlockSpec((1, 1, QB), lambda i: (i, 0, 0)),
        out_shape=jax.ShapeDtypeStruct((nq, 1, QB), jnp.int32),
        scratch_shapes=[
            pltpu.VMEM((d + 1, LPAD), jnp.float32),
            pltpu.VMEM((1, LPAD), jnp.float32),
            pltpu.VMEM((1, LPAD), jnp.int32),
        ],
    )(x, initc, lab2d)
    return labels.reshape(q)


# speedup vs baseline: 2.0695x; 1.0278x over previous
import functools

import jax
import jax.numpy as jnp
from jax.experimental import pallas as pl
from jax.experimental.pallas import tpu as pltpu

QB = 1024
LPAD = 1024


def _cdcl_kernel(nvalid, x_ref, c_ref, lab_ref, out_ref, cm_s, cn_s, code_s):
    @pl.when(pl.program_id(0) == 0)
    def _prep():
        lab = lab_ref[...]
        ksub = jax.lax.broadcasted_iota(
            jnp.int32, (c_ref.shape[0], LPAD), 0)
        oht = (ksub == lab).astype(jnp.float32)
        cm = jax.lax.dot_general(
            c_ref[...], oht, (((0,), (0,)), ((), ())),
            precision=jax.lax.Precision.HIGHEST,
            preferred_element_type=jnp.float32)
        cm_s[...] = cm
        cn = jnp.sum(cm * cm, axis=0, keepdims=True)
        lane = jax.lax.broadcasted_iota(jnp.int32, (1, LPAD), 1)
        cn_s[...] = jnp.where(lane < nvalid, cn, jnp.inf)
        code_s[...] = lane * 2048 + lab

    xb = x_ref[...]
    feac = jnp.concatenate(
        [xb, jnp.ones((xb.shape[0], 1), dtype=xb.dtype)], axis=1)
    nrm = jnp.sqrt(jnp.sum(feac * feac, axis=1, keepdims=True))
    fea = feac / nrm
    dot = jnp.dot(fea, cm_s[...], preferred_element_type=jnp.float32)
    scores = cn_s[...] - 2.0 * dot
    m = jnp.min(scores, axis=1, keepdims=True)
    sel = jnp.where(scores == m, code_s[...], jnp.int32(2**30))
    out_ref[0, 0, :] = jnp.min(sel, axis=1) & 2047


def kernel(x, initc, labelset):
    q, d = x.shape
    l = labelset.shape[0]
    lab2d = jnp.zeros((1, LPAD), dtype=jnp.int32).at[0, :l].set(labelset)

    nq = q // QB
    labels = pl.pallas_call(
        functools.partial(_cdcl_kernel, l),
        grid=(nq,),
        in_specs=[
            pl.BlockSpec((QB, d), lambda i: (i, 0)),
            pl.BlockSpec(initc.shape, lambda i: (0, 0)),
            pl.BlockSpec((1, LPAD), lambda i: (0, 0)),
        ],
        out_specs=pl.BlockSpec((1, 1, QB), lambda i: (i, 0, 0)),
        out_shape=jax.ShapeDtypeStruct((nq, 1, QB), jnp.int32),
        scratch_shapes=[
            pltpu.VMEM((d + 1, LPAD), jnp.float32),
            pltpu.VMEM((1, LPAD), jnp.float32),
            pltpu.VMEM((1, LPAD), jnp.int32),
        ],
    )(x, initc, lab2d)
    return labels.reshape(q)


# speedup vs baseline: 2.0798x; 1.0050x over previous
import functools

import jax
import jax.numpy as jnp
from jax.experimental import pallas as pl
from jax.experimental.pallas import tpu as pltpu

QB = 2048
LPAD = 1024


def _cdcl_kernel(nvalid, x_ref, c_ref, lab_ref, out_ref, cm_s, cn_s, code_s):
    @pl.when(pl.program_id(0) == 0)
    def _prep():
        lab = lab_ref[...]
        ksub = jax.lax.broadcasted_iota(
            jnp.int32, (c_ref.shape[0], LPAD), 0)
        oht = (ksub == lab).astype(jnp.float32)
        cm = jax.lax.dot_general(
            c_ref[...], oht, (((0,), (0,)), ((), ())),
            precision=jax.lax.Precision.HIGHEST,
            preferred_element_type=jnp.float32)
        cm_s[...] = cm
        cn = jnp.sum(cm * cm, axis=0, keepdims=True)
        lane = jax.lax.broadcasted_iota(jnp.int32, (1, LPAD), 1)
        cn_s[...] = jnp.where(lane < nvalid, cn, jnp.inf)
        code_s[...] = lane * 2048 + lab

    xb = x_ref[...]
    feac = jnp.concatenate(
        [xb, jnp.ones((xb.shape[0], 1), dtype=xb.dtype)], axis=1)
    nrm = jnp.sqrt(jnp.sum(feac * feac, axis=1, keepdims=True))
    fea = feac / nrm
    dot = jnp.dot(fea, cm_s[...], preferred_element_type=jnp.float32)
    scores = cn_s[...] - 2.0 * dot
    m = jnp.min(scores, axis=1, keepdims=True)
    sel = jnp.where(scores == m, code_s[...], jnp.int32(2**30))
    out_ref[0, 0, :] = jnp.min(sel, axis=1) & 2047


def kernel(x, initc, labelset):
    q, d = x.shape
    l = labelset.shape[0]
    lab2d = jnp.zeros((1, LPAD), dtype=jnp.int32).at[0, :l].set(labelset)

    nq = q // QB
    labels = pl.pallas_call(
        functools.partial(_cdcl_kernel, l),
        grid=(nq,),
        in_specs=[
            pl.BlockSpec((QB, d), lambda i: (i, 0)),
            pl.BlockSpec(initc.shape, lambda i: (0, 0)),
            pl.BlockSpec((1, LPAD), lambda i: (0, 0)),
        ],
        out_specs=pl.BlockSpec((1, 1, QB), lambda i: (i, 0, 0)),
        out_shape=jax.ShapeDtypeStruct((nq, 1, QB), jnp.int32),
        scratch_shapes=[
            pltpu.VMEM((d + 1, LPAD), jnp.float32),
            pltpu.VMEM((1, LPAD), jnp.float32),
            pltpu.VMEM((1, LPAD), jnp.int32),
        ],
    )(x, initc, lab2d)
    return labels.reshape(q)


# speedup vs baseline: 2.1329x; 1.0255x over previous
import functools

import jax
import jax.numpy as jnp
from jax.experimental import pallas as pl
from jax.experimental.pallas import tpu as pltpu

QB = 2048
LPAD = 1024


def _cdcl_kernel(nvalid, x_ref, c_ref, lab_ref, out_ref, cm_s, cn_s, code_s):
    @pl.when(pl.program_id(0) == 0)
    def _prep():
        lab = jnp.concatenate(
            [lab_ref[...],
             jnp.zeros((1, LPAD - lab_ref.shape[1]), jnp.int32)],
            axis=1)
        ksub = jax.lax.broadcasted_iota(
            jnp.int32, (c_ref.shape[0], LPAD), 0)
        oht = (ksub == lab).astype(jnp.float32)
        cm = jax.lax.dot_general(
            c_ref[...], oht, (((0,), (0,)), ((), ())),
            precision=jax.lax.Precision.HIGHEST,
            preferred_element_type=jnp.float32)
        cm_s[...] = cm
        cn = jnp.sum(cm * cm, axis=0, keepdims=True)
        lane = jax.lax.broadcasted_iota(jnp.int32, (1, LPAD), 1)
        cn_s[...] = jnp.where(lane < nvalid, cn, jnp.inf)
        code_s[...] = lane * 2048 + lab

    xb = x_ref[...]
    feac = jnp.concatenate(
        [xb, jnp.ones((xb.shape[0], 1), dtype=xb.dtype)], axis=1)
    nrm = jnp.sqrt(jnp.sum(feac * feac, axis=1, keepdims=True))
    fea = feac / nrm
    dot = jnp.dot(fea, cm_s[...], preferred_element_type=jnp.float32)
    scores = cn_s[...] - 2.0 * dot
    m = jnp.min(scores, axis=1, keepdims=True)
    sel = jnp.where(scores == m, code_s[...], jnp.int32(2**30))
    out_ref[0, 0, :] = jnp.min(sel, axis=1) & 2047


def kernel(x, initc, labelset):
    q, d = x.shape
    l = labelset.shape[0]
    lab2d = labelset.reshape(1, l)

    nq = q // QB
    labels = pl.pallas_call(
        functools.partial(_cdcl_kernel, l),
        grid=(nq,),
        in_specs=[
            pl.BlockSpec((QB, d), lambda i: (i, 0)),
            pl.BlockSpec(initc.shape, lambda i: (0, 0)),
            pl.BlockSpec((1, l), lambda i: (0, 0)),
        ],
        out_specs=pl.BlockSpec((1, 1, QB), lambda i: (i, 0, 0)),
        out_shape=jax.ShapeDtypeStruct((nq, 1, QB), jnp.int32),
        scratch_shapes=[
            pltpu.VMEM((d + 1, LPAD), jnp.float32),
            pltpu.VMEM((1, LPAD), jnp.float32),
            pltpu.VMEM((1, LPAD), jnp.int32),
        ],
    )(x, initc, lab2d)
    return labels.reshape(q)
